# baseline (speedup 1.0000x reference)
import functools

import jax
import jax.numpy as jnp
from jax import lax
from jax.experimental import pallas as pl
from jax.experimental.pallas import tpu as pltpu
from jax.experimental.pallas import tpu_sc as plsc

_B = 16384
_K = 64
_LANES = 16

_INFO = plsc.get_sparse_core_info()
_NC = _INFO.num_cores
_NS = _INFO.num_subcores
_NW = _NC * _NS
_BPW = _B // _NW
_CHUNK = 128
_NCHUNK = _BPW // _CHUNK


def _body(uidx_hbm, iidx_hbm, t_hbm, ut_hbm, it_hbm, tt_hbm, bias_hbm,
          out_hbm, ue_hbm, ie_hbm,
          idx_u, idx_i, u_rows, i_rows, t_v, tt_v, bias_v, out_v, sem):
  wid = lax.axis_index("s") * _NC + lax.axis_index("c")
  base = wid * _BPW

  pltpu.sync_copy(uidx_hbm.at[pl.ds(wid * _NCHUNK, _NCHUNK)], idx_u)
  pltpu.sync_copy(iidx_hbm.at[pl.ds(wid * _NCHUNK, _NCHUNK)], idx_i)

  copies = []
  for j in range(_NCHUNK):
    copies.append(pltpu.async_copy(
        ut_hbm.at[idx_u.at[j]], u_rows.at[pl.ds(j * _CHUNK, _CHUNK)], sem))
    copies.append(pltpu.async_copy(
        it_hbm.at[idx_i.at[j]], i_rows.at[pl.ds(j * _CHUNK, _CHUNK)], sem))

  pltpu.sync_copy(tt_hbm, tt_v)
  pltpu.sync_copy(bias_hbm, bias_v)
  pltpu.sync_copy(t_hbm, t_v)

  def t_step(i, acc):
    return acc + t_v[pl.ds(i * _LANES, _LANES)]
  n1v = lax.fori_loop(0, _B // _LANES, t_step, jnp.zeros((_LANES,), jnp.int32))
  n1 = jnp.sum(n1v.astype(jnp.float32))
  s0 = jnp.sum(tt_v[0, :] + tt_v[1, :])
  s1 = jnp.sum(tt_v[2, :] + tt_v[3, :])
  scalar = (jnp.float32(_B) - n1) * s0 + n1 * s1 + bias_v[:][0]

  for c in copies:
    c.wait()

  def dot_group(g, carry):
    b0 = g * _LANES
    rows = b0 + lax.iota(jnp.int32, _LANES)
    acc = jnp.full((_LANES,), scalar, jnp.float32)
    for k in range(_K):
      col = jnp.full((_LANES,), k, jnp.int32)
      acc = acc + (plsc.load_gather(u_rows, [rows, col]) *
                   plsc.load_gather(i_rows, [rows, col]))
    out_v[pl.ds(b0, _LANES)] = acc
    return carry
  lax.fori_loop(0, _BPW // _LANES, dot_group, 0)

  pltpu.sync_copy(out_v, out_hbm.at[pl.ds(base, _BPW)])
  pltpu.sync_copy(u_rows, ue_hbm.at[pl.ds(base, _BPW)])
  pltpu.sync_copy(i_rows, ie_hbm.at[pl.ds(base, _BPW)])


@jax.jit
def _sc_forward(uidx, iidx, t, user_table, item_table, tt4, bias):
  mesh = plsc.VectorSubcoreMesh(core_axis_name="c", subcore_axis_name="s")
  call = pl.kernel(
      _body,
      out_type=[
          jax.ShapeDtypeStruct((_B,), jnp.float32),
          jax.ShapeDtypeStruct((_B, _K), jnp.float32),
          jax.ShapeDtypeStruct((_B, _K), jnp.float32),
      ],
      mesh=mesh,
      compiler_params=pltpu.CompilerParams(
          needs_layout_passes=False, use_tc_tiling_on_sc=False),
      scratch_types=[
          pltpu.VMEM((_NCHUNK, _CHUNK), jnp.int32),
          pltpu.VMEM((_NCHUNK, _CHUNK), jnp.int32),
          pltpu.VMEM((_BPW, _K), jnp.float32),
          pltpu.VMEM((_BPW, _K), jnp.float32),
          pltpu.VMEM((_B,), jnp.int32),
          pltpu.VMEM((4, _LANES), jnp.float32),
          pltpu.VMEM((_LANES,), jnp.float32),
          pltpu.VMEM((_BPW,), jnp.float32),
          pltpu.SemaphoreType.DMA,
      ],
  )
  return call(uidx, iidx, t, user_table, item_table, tt4, bias)


def kernel(x, user_table, item_table, treatment_table, bias):
  x = x.astype(jnp.int32)
  uidx = x[:, 0].reshape(_B // _CHUNK, _CHUNK)
  iidx = x[:, 1].reshape(_B // _CHUNK, _CHUNK)
  t = x[:, 2]
  tt4 = treatment_table.reshape(4, _LANES)
  bias16 = jnp.broadcast_to(bias.astype(jnp.float32), (_LANES,))
  out_flat, u_e, i_e = _sc_forward(
      uidx, iidx, t, user_table, item_table, tt4, bias16)
  return (out_flat.reshape(_B, 1), u_e, i_e)


# speedup vs baseline: 1.0135x; 1.0135x over previous
import functools

import jax
import jax.numpy as jnp
from jax import lax
from jax.experimental import pallas as pl
from jax.experimental.pallas import tpu as pltpu
from jax.experimental.pallas import tpu_sc as plsc

_B = 16384
_K = 64
_LANES = 16

_INFO = plsc.get_sparse_core_info()
_NC = _INFO.num_cores
_NS = _INFO.num_subcores
_NW = _NC * _NS
_BPW = _B // _NW
_CHUNK = 128
_NCHUNK = _BPW // _CHUNK
_NGRP = _BPW // _LANES

_PARAMS = pltpu.CompilerParams(
    needs_layout_passes=False, use_tc_tiling_on_sc=False)


def _gather_body(idx2d_hbm, table_hbm, out_hbm, idx_v, rows_v, sem):
  wid = lax.axis_index("s") * _NC + lax.axis_index("c")
  base = wid * _BPW

  pltpu.sync_copy(idx2d_hbm.at[pl.ds(wid * _NCHUNK, _NCHUNK)], idx_v)
  copies = []
  for j in range(_NCHUNK):
    copies.append(pltpu.async_copy(
        table_hbm.at[idx_v.at[j]], rows_v.at[pl.ds(j * _CHUNK, _CHUNK)], sem))
  for c in copies:
    c.wait()
  pltpu.sync_copy(rows_v, out_hbm.at[pl.ds(base, _BPW)])


def _dot_body(ue_hbm, ie_hbm, t_hbm, tt_hbm, bias_hbm, out_hbm,
              u_rows, i_rows, t_v, tt_v, bias_v, out_v):
  wid = lax.axis_index("s") * _NC + lax.axis_index("c")
  base = wid * _BPW

  pltpu.sync_copy(ue_hbm.at[pl.ds(base, _BPW)], u_rows)
  pltpu.sync_copy(ie_hbm.at[pl.ds(base, _BPW)], i_rows)
  pltpu.sync_copy(tt_hbm, tt_v)
  pltpu.sync_copy(bias_hbm, bias_v)
  pltpu.sync_copy(t_hbm, t_v)

  def t_step(i, acc):
    return acc + t_v[pl.ds(i * _LANES, _LANES)]
  n1v = lax.fori_loop(0, _B // _LANES, t_step, jnp.zeros((_LANES,), jnp.int32))
  n1 = jnp.sum(n1v.astype(jnp.float32))
  s0 = jnp.sum(tt_v[pl.ds(0, _LANES)] + tt_v[pl.ds(_LANES, _LANES)])
  s1 = jnp.sum(tt_v[pl.ds(2 * _LANES, _LANES)] + tt_v[pl.ds(3 * _LANES, _LANES)])
  scalar = (jnp.float32(_B) - n1) * s0 + n1 * s1 + bias_v[:][0]

  def dot_group(g, carry):
    b0 = g * _LANES
    rows = b0 + lax.iota(jnp.int32, _LANES)
    acc = jnp.full((_LANES,), scalar, jnp.float32)
    for k in range(_K):
      col = jnp.full((_LANES,), k, jnp.int32)
      acc = acc + (plsc.load_gather(u_rows, [rows, col]) *
                   plsc.load_gather(i_rows, [rows, col]))
    out_v[pl.ds(b0, _LANES)] = acc
    return carry
  lax.fori_loop(0, _NGRP, dot_group, 0)

  pltpu.sync_copy(out_v, out_hbm.at[pl.ds(base, _BPW)])


def _make_gather():
  mesh = plsc.VectorSubcoreMesh(core_axis_name="c", subcore_axis_name="s")
  return pl.kernel(
      _gather_body,
      out_type=jax.ShapeDtypeStruct((_B, _K), jnp.float32),
      mesh=mesh,
      compiler_params=_PARAMS,
      scratch_types=[
          pltpu.VMEM((_NCHUNK, _CHUNK), jnp.int32),
          pltpu.VMEM((_BPW, _K), jnp.float32),
          pltpu.SemaphoreType.DMA,
      ],
  )


@jax.jit
def _sc_forward(uidx2d, iidx2d, t, user_table, item_table, tt, bias16):
  ue_flat = _make_gather()(uidx2d, user_table)
  ie_flat = _make_gather()(iidx2d, item_table)
  mesh = plsc.VectorSubcoreMesh(core_axis_name="c", subcore_axis_name="s")
  dot = pl.kernel(
      _dot_body,
      out_type=jax.ShapeDtypeStruct((_B,), jnp.float32),
      mesh=mesh,
      compiler_params=_PARAMS,
      scratch_types=[
          pltpu.VMEM((_BPW, _K), jnp.float32),
          pltpu.VMEM((_BPW, _K), jnp.float32),
          pltpu.VMEM((_B,), jnp.int32),
          pltpu.VMEM((4 * _LANES,), jnp.float32),
          pltpu.VMEM((_LANES,), jnp.float32),
          pltpu.VMEM((_BPW,), jnp.float32),
      ],
  )
  out_flat = dot(ue_flat, ie_flat, t, tt, bias16)
  return out_flat, ue_flat, ie_flat


def _finish(out_flat, ue, ie):
  return out_flat.reshape(_B, 1), ue, ie


def kernel(x, user_table, item_table, treatment_table, bias):
  x = x.astype(jnp.int32)
  uidx2d = x[:, 0].reshape(_B // _CHUNK, _CHUNK)
  iidx2d = x[:, 1].reshape(_B // _CHUNK, _CHUNK)
  t = x[:, 2]
  tt = treatment_table.reshape(4 * _LANES)
  bias16 = jnp.broadcast_to(bias.astype(jnp.float32), (_LANES,))
  out_flat, ue, ie = _sc_forward(
      uidx2d, iidx2d, t, user_table, item_table, tt, bias16)
  return _finish(out_flat, ue, ie)


# speedup vs baseline: 2.7600x; 2.7232x over previous
import functools

import jax
import jax.numpy as jnp
from jax import lax
from jax.experimental import pallas as pl
from jax.experimental.pallas import tpu as pltpu
from jax.experimental.pallas import tpu_sc as plsc

_B = 16384
_V = 1000000
_K = 64
_LANES = 16

_INFO = plsc.get_sparse_core_info()
_NC = _INFO.num_cores
_NS = _INFO.num_subcores
_NW = _NC * _NS
_BPW = _B // _NW
_NGRP = _BPW // _LANES
_RING = 8
_TAIL = (_V // 128) * 128
_BLOCK_BYTES = _K * 128 * 4

_PARAMS = pltpu.CompilerParams(
    needs_layout_passes=False, disable_bounds_checks=True)


def _gather_body(uidx_hbm, iidx_hbm, ut_hbm, it_hbm, ue_hbm, ie_hbm,
                 idx_u, idx_i,
                 s0, s1, s2, s3, s4, s5, s6, s7,
                 r0, r1, r2, r3, r4, r5, r6, r7,
                 sem, sem2):
  slabs = (s0, s1, s2, s3, s4, s5, s6, s7)
  rowbufs = (r0, r1, r2, r3, r4, r5, r6, r7)
  wid = lax.axis_index("s") * _NC + lax.axis_index("c")
  base = wid * _BPW

  pltpu.sync_copy(uidx_hbm.at[pl.ds(base, _BPW)], idx_u)
  pltpu.sync_copy(iidx_hbm.at[pl.ds(base, _BPW)], idx_i)

  rows4 = [k * _LANES + lax.iota(jnp.int32, _LANES) for k in range(4)]

  def fetch(table, r, slab):
    c0 = pl.multiple_of((r >> 7) * 128, 128)
    pltpu.async_copy(table.at[:, pl.ds(c0, 128)], slab, sem)
    return r - c0

  def drain_extract(slot, rr, out_hbm, b):
    pltpu.make_async_copy(ut_hbm.at[:, pl.ds(0, 128)], slabs[slot], sem).wait()
    col = jnp.full((_LANES,), rr, jnp.int32)
    rb = rowbufs[slot]
    for k in range(4):
      rb[pl.ds(k * _LANES, _LANES)] = plsc.load_gather(
          slabs[slot], [rows4[k], col])
    pltpu.async_copy(rb, out_hbm.at[pl.ds((base + b) * _K, _K)], sem2)

  def prev_meta(g, l):
    lp = l - 8 if l >= 8 else l + 24
    gp = g if l >= 8 else g - 1
    if lp < 16:
      return ue_hbm, gp * _LANES + lp
    return ie_hbm, gp * _LANES + (lp - 16)

  def body(g, carry):
    rrs = list(carry)
    vu = idx_u[pl.ds(g * _LANES, _LANES)]
    vi = idx_i[pl.ds(g * _LANES, _LANES)]
    new_rrs = []
    for l in range(32):
      slot = l % _RING
      out_hbm, b = prev_meta(g, l)
      rr_prev = rrs[l] if l < 8 else new_rrs[l - 8]

      def de(out_hbm=out_hbm, b=b, rr_prev=rr_prev, slot=slot):
        drain_extract(slot, rr_prev, out_hbm, b)
        pltpu.make_async_copy(ut_hbm.at[0, pl.ds(0, _K)],
                              rowbufs[slot], sem2).wait()
      if l < 8:
        lax.cond(g > 0, de, lambda: None)
      else:
        de()

      if l < 16:
        new_rrs.append(fetch(ut_hbm, vu[l], slabs[slot]))
      else:
        new_rrs.append(fetch(it_hbm, vi[l - 16], slabs[slot]))
    return tuple(new_rrs[24:])

  carry0 = tuple(jnp.int32(0) for _ in range(_RING))
  carry = lax.fori_loop(0, _NGRP, body, carry0)

  for l in range(8):
    slot = l % _RING
    b = (_NGRP - 1) * _LANES + 8 + l
    drain_extract(slot, carry[l], ie_hbm, b)
    pltpu.make_async_copy(ut_hbm.at[0, pl.ds(0, _K)],
                          rowbufs[slot], sem2).wait()


def _dot_body(ue_hbm, ie_hbm, t_hbm, tt_hbm, bias_hbm, out_hbm,
              u_flat, i_flat, t_v, tt_v, bias_v, out_v):
  wid = lax.axis_index("s") * _NC + lax.axis_index("c")
  base = wid * _BPW

  pltpu.sync_copy(ue_hbm.at[pl.ds(base * _K, _BPW * _K)], u_flat)
  pltpu.sync_copy(ie_hbm.at[pl.ds(base * _K, _BPW * _K)], i_flat)
  pltpu.sync_copy(tt_hbm, tt_v)
  pltpu.sync_copy(bias_hbm, bias_v)
  pltpu.sync_copy(t_hbm, t_v)

  def t_step(i, acc):
    return acc + t_v[pl.ds(i * _LANES, _LANES)]
  n1v = lax.fori_loop(0, _B // _LANES, t_step, jnp.zeros((_LANES,), jnp.int32))
  n1 = jnp.sum(n1v.astype(jnp.float32))
  s0 = jnp.sum(tt_v[pl.ds(0, _LANES)] + tt_v[pl.ds(_LANES, _LANES)])
  s1 = jnp.sum(tt_v[pl.ds(2 * _LANES, _LANES)] + tt_v[pl.ds(3 * _LANES, _LANES)])
  scalar = (jnp.float32(_B) - n1) * s0 + n1 * s1 + bias_v[:][0]

  iota64 = lax.iota(jnp.int32, _LANES) * _K
  def dot_group(g, carry):
    bidx = g * (_LANES * _K) + iota64
    acc = jnp.full((_LANES,), scalar, jnp.float32)
    for k in range(_K):
      idx_k = bidx + k
      acc = acc + (plsc.load_gather(u_flat, [idx_k]) *
                   plsc.load_gather(i_flat, [idx_k]))
    out_v[pl.ds(g * _LANES, _LANES)] = acc
    return carry
  lax.fori_loop(0, _NGRP, dot_group, 0)

  pltpu.sync_copy(out_v, out_hbm.at[pl.ds(base, _BPW)])


@jax.jit
def _sc_forward(uidx, iidx, t, ut_t, it_t, tt, bias16):
  mesh = plsc.VectorSubcoreMesh(core_axis_name="c", subcore_axis_name="s")
  gather = pl.kernel(
      _gather_body,
      out_type=[
          jax.ShapeDtypeStruct((_B * _K,), jnp.float32),
          jax.ShapeDtypeStruct((_B * _K,), jnp.float32),
      ],
      mesh=mesh,
      compiler_params=_PARAMS,
      scratch_types=(
          [pltpu.VMEM((_BPW,), jnp.int32)] * 2
          + [pltpu.VMEM((_K, 128), jnp.float32)] * _RING
          + [pltpu.VMEM((_K,), jnp.float32)] * _RING
          + [pltpu.SemaphoreType.DMA] * 2
      ),
  )
  ue_flat, ie_flat = gather(uidx, iidx, ut_t, it_t)

  dot = pl.kernel(
      _dot_body,
      out_type=jax.ShapeDtypeStruct((_B,), jnp.float32),
      mesh=mesh,
      compiler_params=_PARAMS,
      scratch_types=[
          pltpu.VMEM((_BPW * _K,), jnp.float32),
          pltpu.VMEM((_BPW * _K,), jnp.float32),
          pltpu.VMEM((_B,), jnp.int32),
          pltpu.VMEM((4 * _LANES,), jnp.float32),
          pltpu.VMEM((_LANES,), jnp.float32),
          pltpu.VMEM((_BPW,), jnp.float32),
      ],
  )
  out_flat = dot(ue_flat, ie_flat, t, tt, bias16)
  return out_flat, ue_flat, ie_flat


def kernel(x, user_table, item_table, treatment_table, bias):
  x = x.astype(jnp.int32)
  uidx = x[:, 0]
  iidx = x[:, 1]
  t = x[:, 2]
  tt = treatment_table.reshape(4 * _LANES)
  bias16 = jnp.broadcast_to(bias.astype(jnp.float32), (_LANES,))
  out_flat, ue_flat, ie_flat = _sc_forward(
      uidx, iidx, t, user_table.T, item_table.T, tt, bias16)
  return (out_flat.reshape(_B, 1),
          ue_flat.reshape(_B, _K),
          ie_flat.reshape(_B, _K))


# speedup vs baseline: 2.7992x; 1.0142x over previous
import functools

import jax
import jax.numpy as jnp
from jax import lax
from jax.experimental import pallas as pl
from jax.experimental.pallas import tpu as pltpu
from jax.experimental.pallas import tpu_sc as plsc

_B = 16384
_V = 1000000
_K = 64
_LANES = 16

_INFO = plsc.get_sparse_core_info()
_NC = _INFO.num_cores
_NS = _INFO.num_subcores
_NW = _NC * _NS
_BPW = _B // _NW
_NGRP = _BPW // _LANES
_RING = 4
_BLOCK_BYTES = _K * 128 * 4

_PARAMS = pltpu.CompilerParams(
    needs_layout_passes=False, disable_bounds_checks=True)


def _body(uidx_hbm, iidx_hbm, t_hbm, ut_hbm, it_hbm, tt_hbm, bias_hbm,
          out_hbm, ue_hbm, ie_hbm,
          idx_u, idx_i, u_rows, i_ring, t_v, tt_v, bias_v, out_v,
          s0, s1, s2, s3, sem, sem2):
  slabs = (s0, s1, s2, s3)
  wid = lax.axis_index("s") * _NC + lax.axis_index("c")
  base = wid * _BPW

  pltpu.sync_copy(uidx_hbm.at[pl.ds(base, _BPW)], idx_u)
  pltpu.sync_copy(iidx_hbm.at[pl.ds(base, _BPW)], idx_i)

  rows4 = [k * _LANES + lax.iota(jnp.int32, _LANES) for k in range(4)]
  lane_iota = lax.iota(jnp.int32, _LANES)

  def fetch(table, r, slab):
    c0 = pl.multiple_of((r >> 7) * 128, 128)
    pltpu.async_copy(table.at[:, pl.ds(c0, 128)], slab, sem)
    return r - c0

  def drain_fetch(slot):
    pltpu.make_async_copy(ut_hbm.at[:, pl.ds(0, 128)], slabs[slot],
                          sem).wait()

  def extract(slot, rr):
    col = jnp.full((_LANES,), rr, jnp.int32)
    return [plsc.load_gather(slabs[slot], [rows4[k], col]) for k in range(4)]

  def drain_iouts(n):
    pltpu.make_async_copy(ie_hbm.at[pl.ds(0, n * _K)],
                          i_ring.at[pl.ds(0, n * _K)], sem2).wait()

  def do_i_row(slot, rr_sel, b_local, lane, dv):
    drain_fetch(slot)
    vecs = extract(slot, rr_sel)
    im = (b_local & 31) * _K
    acc = u_rows[b_local, pl.ds(0, _LANES)] * vecs[0]
    for k in range(4):
      i_ring[pl.ds(im + k * _LANES, _LANES)] = vecs[k]
      if k:
        acc = acc + u_rows[b_local, pl.ds(k * _LANES, _LANES)] * vecs[k]
    d = jnp.sum(acc)
    pltpu.async_copy(i_ring.at[pl.ds(im, _K)],
                     ie_hbm.at[pl.ds((base + b_local) * _K, _K)], sem2)
    return jnp.where(lane_iota == lane, d, dv)

  def body(g, carry):
    rrs = list(carry[:_RING])
    dotvec = carry[_RING]

    lax.switch(jnp.clip(g, 0, 2),
               [lambda: None, lambda: drain_iouts(12), lambda: drain_iouts(16)])

    vu = idx_u[pl.ds(g * _LANES, _LANES)]
    vi = idx_i[pl.ds(g * _LANES, _LANES)]
    new_rrs = []
    for l in range(32):
      slot = l % _RING
      if l < _RING:
        lp, gb = l + 32 - _RING, g - 1
      else:
        lp, gb = l - _RING, g
      is_u = lp < 16
      b_local = gb * _LANES + (lp if is_u else lp - _LANES)
      rr_sel = rrs[l] if l < _RING else new_rrs[l - _RING]

      if is_u:
        def de_u(slot=slot, rr_sel=rr_sel, b_local=b_local):
          drain_fetch(slot)
          vecs = extract(slot, rr_sel)
          for k in range(4):
            u_rows[b_local, pl.ds(k * _LANES, _LANES)] = vecs[k]
        de_u()
      else:
        lane = lp - _LANES
        def de_i(slot=slot, rr_sel=rr_sel, b_local=b_local, lane=lane,
                 dv=dotvec):
          return do_i_row(slot, rr_sel, b_local, lane, dv)
        if l < _RING:
          dotvec = lax.cond(g > 0, de_i, lambda dv=dotvec: dv)
        else:
          dotvec = de_i()

      if l == _RING - 1:
        def store_dots(dv=dotvec, g=g):
          out_v[pl.ds((g - 1) * _LANES, _LANES)] = dv
        lax.cond(g > 0, store_dots, lambda: None)

      if l < 16:
        new_rrs.append(fetch(ut_hbm, vu[l], slabs[slot]))
      else:
        new_rrs.append(fetch(it_hbm, vi[l - 16], slabs[slot]))
    return (*new_rrs[32 - _RING:], dotvec)

  carry0 = tuple(jnp.int32(0) for _ in range(_RING)) + (
      jnp.zeros((_LANES,), jnp.float32),)
  carry = lax.fori_loop(0, _NGRP, body, carry0)
  dotvec = carry[_RING]

  for l in range(_RING):
    b_local = _BPW - _RING + l
    dotvec = do_i_row(l % _RING, carry[l], b_local, 16 - _RING + l, dotvec)
  out_v[pl.ds((_NGRP - 1) * _LANES, _LANES)] = dotvec
  drain_iouts(16 + _RING)

  pltpu.sync_copy(tt_hbm, tt_v)
  pltpu.sync_copy(bias_hbm, bias_v)
  def t_step(i, acc):
    return acc + t_v[pl.ds(i * _LANES, _LANES)]
  n1v = jnp.zeros((_LANES,), jnp.int32)
  for c in range(4):
    pltpu.sync_copy(t_hbm.at[pl.ds(c * 4096, 4096)], t_v)
    n1v = lax.fori_loop(0, 4096 // _LANES, t_step, n1v)
  n1 = jnp.sum(n1v.astype(jnp.float32))
  sm0 = jnp.sum(tt_v[pl.ds(0, _LANES)] + tt_v[pl.ds(_LANES, _LANES)])
  sm1 = jnp.sum(tt_v[pl.ds(2 * _LANES, _LANES)] + tt_v[pl.ds(3 * _LANES, _LANES)])
  scalar = (jnp.float32(_B) - n1) * sm0 + n1 * sm1 + bias_v[:][0]
  def add_s(j, carry):
    out_v[pl.ds(j * _LANES, _LANES)] = (
        out_v[pl.ds(j * _LANES, _LANES)] + scalar)
    return carry
  lax.fori_loop(0, _NGRP, add_s, 0)

  pltpu.sync_copy(out_v, out_hbm.at[pl.ds(base, _BPW)])
  pltpu.sync_copy(u_rows, ue_hbm.at[pl.ds(base, _BPW)])


@jax.jit
def _sc_forward(uidx, iidx, t, ut_t, it_t, tt, bias16):
  mesh = plsc.VectorSubcoreMesh(core_axis_name="c", subcore_axis_name="s")
  call = pl.kernel(
      _body,
      out_type=[
          jax.ShapeDtypeStruct((_B,), jnp.float32),
          jax.ShapeDtypeStruct((_B, _K), jnp.float32),
          jax.ShapeDtypeStruct((_B * _K,), jnp.float32),
      ],
      mesh=mesh,
      compiler_params=_PARAMS,
      scratch_types=(
          [pltpu.VMEM((_BPW,), jnp.int32)] * 2
          + [pltpu.VMEM((_BPW, _K), jnp.float32)]
          + [pltpu.VMEM((32 * _K,), jnp.float32)]
          + [pltpu.VMEM((4096,), jnp.int32)]
          + [pltpu.VMEM((4 * _LANES,), jnp.float32)]
          + [pltpu.VMEM((_LANES,), jnp.float32)]
          + [pltpu.VMEM((_BPW,), jnp.float32)]
          + [pltpu.VMEM((_K, 128), jnp.float32)] * _RING
          + [pltpu.SemaphoreType.DMA] * 2
      ),
  )
  return call(uidx, iidx, t, ut_t, it_t, tt, bias16)


def kernel(x, user_table, item_table, treatment_table, bias):
  x = x.astype(jnp.int32)
  uidx = x[:, 0]
  iidx = x[:, 1]
  t = x[:, 2]
  tt = treatment_table.reshape(4 * _LANES)
  bias16 = jnp.broadcast_to(bias.astype(jnp.float32), (_LANES,))
  out_flat, ue, ie_flat = _sc_forward(
      uidx, iidx, t, user_table.T, item_table.T, tt, bias16)
  return (out_flat.reshape(_B, 1), ue, ie_flat.reshape(_B, _K))


# speedup vs baseline: 2.8800x; 1.0289x over previous
import functools

import jax
import jax.numpy as jnp
from jax import lax
from jax.experimental import pallas as pl
from jax.experimental.pallas import tpu as pltpu
from jax.experimental.pallas import tpu_sc as plsc

_B = 16384
_V = 1000000
_K = 64
_LANES = 16

_INFO = plsc.get_sparse_core_info()
_NC = _INFO.num_cores
_NS = _INFO.num_subcores
_NW = _NC * _NS
_BPW = _B // _NW
_NGRP = _BPW // _LANES
_RING = 8
_BLOCK_BYTES = _K * 128 * 4

_PARAMS = pltpu.CompilerParams(
    needs_layout_passes=False, disable_bounds_checks=True)


def _body(uidx_hbm, iidx_hbm, t_hbm, ut_hbm, it_hbm, tt_hbm, bias_hbm,
          out_hbm, ue_hbm, ie_hbm,
          idx_u, idx_i, u_ring, i_ring, t_v, tt_v, bias_v, out_v,
          s0, s1, s2, s3, s4, s5, s6, s7, sem, sem2):
  slabs = (s0, s1, s2, s3, s4, s5, s6, s7)
  wid = lax.axis_index("s") * _NC + lax.axis_index("c")
  base = wid * _BPW

  pltpu.sync_copy(uidx_hbm.at[pl.ds(base, _BPW)], idx_u)
  pltpu.sync_copy(iidx_hbm.at[pl.ds(base, _BPW)], idx_i)

  rows4 = [k * _LANES + lax.iota(jnp.int32, _LANES) for k in range(4)]
  lane_iota = lax.iota(jnp.int32, _LANES)

  def fetch(table, r, slab):
    c0 = pl.multiple_of((r >> 7) * 128, 128)
    pltpu.async_copy(table.at[:, pl.ds(c0, 128)], slab, sem)
    return r - c0

  def drain_fetch(slot):
    pltpu.make_async_copy(ut_hbm.at[:, pl.ds(0, 128)], slabs[slot],
                          sem).wait()

  def extract(slot, rr):
    col = jnp.full((_LANES,), rr, jnp.int32)
    return [plsc.load_gather(slabs[slot], [rows4[k], col]) for k in range(4)]

  def drain_iouts(n):
    pltpu.make_async_copy(ie_hbm.at[pl.ds(0, n * _K)],
                          i_ring.at[pl.ds(0, n * _K)], sem2).wait()

  def do_i_row(slot, rr_sel, b_local, lane, dv):
    drain_fetch(slot)
    vecs = extract(slot, rr_sel)
    im = (b_local & 31) * _K
    acc = u_ring[pl.ds(im, _LANES)] * vecs[0]
    for k in range(4):
      i_ring[pl.ds(im + k * _LANES, _LANES)] = vecs[k]
      if k:
        acc = acc + u_ring[pl.ds(im + k * _LANES, _LANES)] * vecs[k]
    d = jnp.sum(acc)
    pltpu.async_copy(i_ring.at[pl.ds(im, _K)],
                     ie_hbm.at[pl.ds((base + b_local) * _K, _K)], sem2)
    return jnp.where(lane_iota == lane, d, dv)

  def body(g, carry):
    rrs = list(carry[:_RING])
    dotvec = carry[_RING]

    lax.switch(jnp.clip(g, 0, 2),
               [lambda: None, lambda: drain_iouts(24), lambda: drain_iouts(32)])

    vu = idx_u[pl.ds(g * _LANES, _LANES)]
    vi = idx_i[pl.ds(g * _LANES, _LANES)]
    new_rrs = []
    for l in range(32):
      slot = l % _RING
      if l < _RING:
        lp, gb = l + 32 - _RING, g - 1
      else:
        lp, gb = l - _RING, g
      is_u = lp < 16
      b_local = gb * _LANES + (lp if is_u else lp - _LANES)
      rr_sel = rrs[l] if l < _RING else new_rrs[l - _RING]

      if is_u:
        def de_u(slot=slot, rr_sel=rr_sel, b_local=b_local):
          um = (b_local & 31) * _K
          drain_fetch(slot)
          vecs = extract(slot, rr_sel)
          for k in range(4):
            u_ring[pl.ds(um + k * _LANES, _LANES)] = vecs[k]
          pltpu.async_copy(u_ring.at[pl.ds(um, _K)],
                           ue_hbm.at[pl.ds((base + b_local) * _K, _K)], sem2)
        de_u()
      else:
        lane = lp - _LANES
        def de_i(slot=slot, rr_sel=rr_sel, b_local=b_local, lane=lane,
                 dv=dotvec):
          return do_i_row(slot, rr_sel, b_local, lane, dv)
        if l < _RING:
          dotvec = lax.cond(g > 0, de_i, lambda dv=dotvec: dv)
        else:
          dotvec = de_i()

      if l == _RING - 1:
        def store_dots(dv=dotvec, g=g):
          out_v[pl.ds((g - 1) * _LANES, _LANES)] = dv
        lax.cond(g > 0, store_dots, lambda: None)

      if l < 16:
        new_rrs.append(fetch(ut_hbm, vu[l], slabs[slot]))
      else:
        new_rrs.append(fetch(it_hbm, vi[l - 16], slabs[slot]))
    return (*new_rrs[32 - _RING:], dotvec)

  carry0 = tuple(jnp.int32(0) for _ in range(_RING)) + (
      jnp.zeros((_LANES,), jnp.float32),)
  carry = lax.fori_loop(0, _NGRP, body, carry0)
  dotvec = carry[_RING]

  for l in range(_RING):
    b_local = _BPW - _RING + l
    dotvec = do_i_row(l % _RING, carry[l], b_local, 16 - _RING + l, dotvec)
  out_v[pl.ds((_NGRP - 1) * _LANES, _LANES)] = dotvec
  drain_iouts(32)
  drain_iouts(_RING)

  pltpu.sync_copy(tt_hbm, tt_v)
  pltpu.sync_copy(bias_hbm, bias_v)
  def t_step(i, acc):
    return acc + t_v[pl.ds(i * _LANES, _LANES)]
  n1v = jnp.zeros((_LANES,), jnp.int32)
  for c in range(4):
    pltpu.sync_copy(t_hbm.at[pl.ds(c * 4096, 4096)], t_v)
    n1v = lax.fori_loop(0, 4096 // _LANES, t_step, n1v)
  n1 = jnp.sum(n1v.astype(jnp.float32))
  sm0 = jnp.sum(tt_v[pl.ds(0, _LANES)] + tt_v[pl.ds(_LANES, _LANES)])
  sm1 = jnp.sum(tt_v[pl.ds(2 * _LANES, _LANES)] + tt_v[pl.ds(3 * _LANES, _LANES)])
  scalar = (jnp.float32(_B) - n1) * sm0 + n1 * sm1 + bias_v[:][0]
  def add_s(j, carry):
    out_v[pl.ds(j * _LANES, _LANES)] = (
        out_v[pl.ds(j * _LANES, _LANES)] + scalar)
    return carry
  lax.fori_loop(0, _NGRP, add_s, 0)

  pltpu.sync_copy(out_v, out_hbm.at[pl.ds(base, _BPW)])


@jax.jit
def _sc_forward(uidx, iidx, t, ut_t, it_t, tt, bias16):
  mesh = plsc.VectorSubcoreMesh(core_axis_name="c", subcore_axis_name="s")
  call = pl.kernel(
      _body,
      out_type=[
          jax.ShapeDtypeStruct((_B,), jnp.float32),
          jax.ShapeDtypeStruct((_B * _K,), jnp.float32),
          jax.ShapeDtypeStruct((_B * _K,), jnp.float32),
      ],
      mesh=mesh,
      compiler_params=_PARAMS,
      scratch_types=(
          [pltpu.VMEM((_BPW,), jnp.int32)] * 2
          + [pltpu.VMEM((32 * _K,), jnp.float32)]
          + [pltpu.VMEM((32 * _K,), jnp.float32)]
          + [pltpu.VMEM((4096,), jnp.int32)]
          + [pltpu.VMEM((4 * _LANES,), jnp.float32)]
          + [pltpu.VMEM((_LANES,), jnp.float32)]
          + [pltpu.VMEM((_BPW,), jnp.float32)]
          + [pltpu.VMEM((_K, 128), jnp.float32)] * _RING
          + [pltpu.SemaphoreType.DMA] * 2
      ),
  )
  return call(uidx, iidx, t, ut_t, it_t, tt, bias16)


def kernel(x, user_table, item_table, treatment_table, bias):
  x = x.astype(jnp.int32)
  uidx = x[:, 0]
  iidx = x[:, 1]
  t = x[:, 2]
  tt = treatment_table.reshape(4 * _LANES)
  bias16 = jnp.broadcast_to(bias.astype(jnp.float32), (_LANES,))
  out_flat, ue_flat, ie_flat = _sc_forward(
      uidx, iidx, t, user_table.T, item_table.T, tt, bias16)
  return (out_flat.reshape(_B, 1), ue_flat.reshape(_B, _K),
          ie_flat.reshape(_B, _K))
